# baseline (speedup 1.0000x reference)
import functools
import jax
import jax.numpy as jnp
from jax import lax
from jax.experimental import pallas as pl
from jax.experimental.pallas import tpu as pltpu
from jax.experimental.pallas import tpu_sc as plsc

N = 10000
E = 320000
D = 128

NC, NS = 2, 16
CHUNK = 128
EB = 40
TB = 64
B0, B1 = 4, 0
E_PAD = TB * EB * CHUNK
NPAD = 10240
RPT = NPAD // NS

_mesh = plsc.VectorSubcoreMesh(core_axis_name="c", subcore_axis_name="s")


@functools.partial(
    pl.kernel,
    out_type=jax.ShapeDtypeStruct((NC, NPAD, D), jnp.float32),
    mesh=_mesh,
    scratch_types=[
        pltpu.VMEM((EB, CHUNK), jnp.int32),
        pltpu.VMEM((EB, CHUNK), jnp.int32),
        pltpu.VMEM((EB, CHUNK), jnp.float32),
        pltpu.VMEM((CHUNK, D), jnp.float32),
        pltpu.VMEM((CHUNK, D), jnp.float32),
        pltpu.VMEM_SHARED((NPAD, D), jnp.float32),
        pltpu.SemaphoreType.DMA,
        pltpu.SemaphoreType.DMA,
    ],
)
def _aggregate(h_hbm, src_hbm, dst_hbm, ew_hbm, out_hbm,
               src_v, dst_v, ew_v, rows0_v, rows1_v, acc_sh, gsem, ssem):
    c = lax.axis_index("c")
    s = lax.axis_index("s")

    def _zero_stage(i, _):
        for k in range(D // 16):
            rows0_v[i, pl.ds(k * 16, 16)] = jnp.zeros((16,), jnp.float32)
        return 0
    lax.fori_loop(0, CHUNK, _zero_stage, 0)
    for i in range(RPT // CHUNK):
        pltpu.sync_copy(rows0_v, acc_sh.at[pl.ds(s * RPT + i * CHUNK, CHUNK)])
    plsc.subcore_barrier()

    def _scale(rows_v, ew_row):
        def _grp(g, _):
            ewv = ew_v[ew_row, pl.ds(g * 16, 16)]
            for t in range(16):
                wgt = ewv[t]
                r = g * 16 + t
                for k in range(D // 16):
                    sl = pl.ds(k * 16, 16)
                    rows_v[r, sl] = rows_v[r, sl] * wgt
            return 0
        lax.fori_loop(0, CHUNK // 16, _grp, 0)

    def _wait_gather(rows_v):
        pltpu.make_async_copy(h_hbm.at[src_v.at[0]], rows_v, gsem).wait()

    def _wait_scatter(rows_v):
        pltpu.make_async_copy(rows_v, acc_sh.at[dst_v.at[0]], ssem).wait()

    nb = lax.select(c == 0, B0, B1)
    base = lax.select(c == 0, s * B0, NS * B0 + s * B1)

    def _block(b, _):
        blk = base + b
        pltpu.sync_copy(src_hbm.at[blk], src_v)
        pltpu.sync_copy(dst_hbm.at[blk], dst_v)
        pltpu.sync_copy(ew_hbm.at[blk], ew_v)

        pltpu.async_copy(h_hbm.at[src_v.at[0]], rows0_v, gsem)

        def _pair(p, _):
            j0 = 2 * p
            j1 = 2 * p + 1
            _wait_gather(rows0_v)

            @pl.when(p > 0)
            def _():
                _wait_scatter(rows1_v)
            pltpu.async_copy(h_hbm.at[src_v.at[j1]], rows1_v, gsem)
            _scale(rows0_v, j0)
            pltpu.async_copy(rows0_v, acc_sh.at[dst_v.at[j0]], ssem,
                             add=True)
            _wait_gather(rows1_v)
            _scale(rows1_v, j1)
            _wait_scatter(rows0_v)

            @pl.when(p < EB // 2 - 1)
            def _():
                pltpu.async_copy(h_hbm.at[src_v.at[j0 + 2]], rows0_v, gsem)
            pltpu.async_copy(rows1_v, acc_sh.at[dst_v.at[j1]], ssem,
                             add=True)
            return 0
        lax.fori_loop(0, EB // 2, _pair, 0)
        _wait_scatter(rows1_v)
        return 0
    lax.fori_loop(0, nb, _block, 0)
    plsc.subcore_barrier()

    pltpu.sync_copy(acc_sh.at[pl.ds(s * RPT, RPT)],
                    out_hbm.at[c, pl.ds(s * RPT, RPT)])


def _prelu(x, a):
    return jnp.where(x >= 0.0, x, a * x)


def _tc1_body(agg_ref, w1_ref, b1_ref, w2_ref, a1_ref, h2_ref):
    agg = agg_ref[0] + agg_ref[1]
    x1 = jnp.dot(agg, w1_ref[...], preferred_element_type=jnp.float32)
    x1 = _prelu(x1 + b1_ref[...], a1_ref[0, 0])
    h2_ref[...] = jnp.dot(x1, w2_ref[...], preferred_element_type=jnp.float32)


def _tc2_body(agg_ref, b2_ref, a2_ref, seq_ref, wfc1_ref, wfc2_ref,
              wfc3_ref, wfc4_ref, a3_ref, x2_ref, feat_ref):
    agg = agg_ref[0] + agg_ref[1]
    x2 = _prelu(agg + b2_ref[...], a2_ref[0, 0])
    x2_ref[...] = x2
    f = jnp.dot(x2, wfc1_ref[...], preferred_element_type=jnp.float32)
    sproj = jnp.dot(seq_ref[...], wfc2_ref[...],
                    preferred_element_type=jnp.float32)
    u = (jnp.dot(sproj, wfc3_ref[0:256, :], preferred_element_type=jnp.float32)
         + jnp.dot(f, wfc3_ref[256:512, :], preferred_element_type=jnp.float32))
    feat_ref[...] = _prelu(
        jnp.dot(u, wfc4_ref[...], preferred_element_type=jnp.float32),
        a3_ref[0, 0])


_R = 400


def _full(shape):
    return pl.BlockSpec(shape, lambda i: (0,) * len(shape))


def _tc1(agg, w1, b1, w2, a1):
    return pl.pallas_call(
        _tc1_body,
        grid=(N // _R,),
        in_specs=[
            pl.BlockSpec((NC, _R, D), lambda i: (0, i, 0)),
            _full((D, 2 * D)), _full((1, 2 * D)), _full((2 * D, D)),
            _full((1, 1)),
        ],
        out_specs=pl.BlockSpec((_R, D), lambda i: (i, 0)),
        out_shape=jax.ShapeDtypeStruct((N, D), jnp.float32),
    )(agg, w1, b1, w2, a1)


def _tc2(agg, b2, a2, seq, wfc1, wfc2, wfc3, wfc4, a3):
    return pl.pallas_call(
        _tc2_body,
        grid=(N // _R,),
        in_specs=[
            pl.BlockSpec((NC, _R, D), lambda i: (0, i, 0)),
            _full((1, D)), _full((1, 1)),
            pl.BlockSpec((_R, D), lambda i: (i, 0)),
            _full((D, 2 * D)), _full((D, 2 * D)), _full((4 * D, D)),
            _full((D, D)), _full((1, 1)),
        ],
        out_specs=[
            pl.BlockSpec((_R, D), lambda i: (i, 0)),
            pl.BlockSpec((_R, D), lambda i: (i, 0)),
        ],
        out_shape=[
            jax.ShapeDtypeStruct((N, D), jnp.float32),
            jax.ShapeDtypeStruct((N, D), jnp.float32),
        ],
    )(agg, b2, a2, seq, wfc1, wfc2, wfc3, wfc4, a3)


def kernel(seq, edge_index, edge_weight, W1, b1, W2, b2, a1, a2, a3,
           Wfc1, Wfc2, Wfc3, Wfc4):
    pad = E_PAD - E
    src = jnp.concatenate(
        [edge_index[0].astype(jnp.int32), jnp.zeros((pad,), jnp.int32)])
    dst = jnp.concatenate(
        [edge_index[1].astype(jnp.int32),
         N + (jnp.arange(pad, dtype=jnp.int32) % (NPAD - N))])
    ew = jnp.concatenate(
        [edge_weight.astype(jnp.float32), jnp.zeros((pad,), jnp.float32)])
    src = src.reshape(TB, EB, CHUNK)
    dst = dst.reshape(TB, EB, CHUNK)
    ew = ew.reshape(TB, EB, CHUNK)

    agg1 = _aggregate(seq, src, dst, ew)
    h2 = _tc1(agg1, W1, b1.reshape(1, -1), W2,
              jnp.asarray(a1, jnp.float32).reshape(1, 1))
    agg2 = _aggregate(h2, src, dst, ew)
    x2, feat1 = _tc2(agg2, b2.reshape(1, -1),
                     jnp.asarray(a2, jnp.float32).reshape(1, 1),
                     seq, Wfc1, Wfc2, Wfc3, Wfc4,
                     jnp.asarray(a3, jnp.float32).reshape(1, 1))
    return (x2, feat1)


# speedup vs baseline: 1.2570x; 1.2570x over previous
import functools
import jax
import jax.numpy as jnp
from jax import lax
from jax.experimental import pallas as pl
from jax.experimental.pallas import tpu as pltpu
from jax.experimental.pallas import tpu_sc as plsc

N = 10000
E = 320000
D = 128

NC, NS = 2, 16
CHUNK = 128
EB = 20
TB = 128
B0, B1 = 5, 3
E_PAD = TB * EB * CHUNK
NPAD = 10240
RPT = NPAD // NS

_mesh = plsc.VectorSubcoreMesh(core_axis_name="c", subcore_axis_name="s")


@functools.partial(
    pl.kernel,
    out_type=jax.ShapeDtypeStruct((NC, NPAD, D), jnp.float32),
    mesh=_mesh,
    scratch_types=[
        pltpu.VMEM((EB, CHUNK), jnp.int32),
        pltpu.VMEM((EB, CHUNK), jnp.int32),
        pltpu.VMEM((EB, CHUNK), jnp.float32),
        pltpu.VMEM((CHUNK, D), jnp.float32),
        pltpu.VMEM((CHUNK, D), jnp.float32),
        pltpu.VMEM_SHARED((NPAD, D), jnp.float32),
        pltpu.SemaphoreType.DMA,
        pltpu.SemaphoreType.DMA,
    ],
)
def _aggregate(h_hbm, src_hbm, dst_hbm, ew_hbm, out_hbm,
               src_v, dst_v, ew_v, rows0_v, rows1_v, acc_sh, gsem, ssem):
    c = lax.axis_index("c")
    s = lax.axis_index("s")

    def _zero_stage(i, _):
        for k in range(D // 16):
            rows0_v[i, pl.ds(k * 16, 16)] = jnp.zeros((16,), jnp.float32)
        return 0
    lax.fori_loop(0, CHUNK, _zero_stage, 0)
    for i in range(RPT // CHUNK):
        pltpu.sync_copy(rows0_v, acc_sh.at[pl.ds(s * RPT + i * CHUNK, CHUNK)])
    plsc.subcore_barrier()

    def _scale(rows_v, ew_row):
        def _grp(g, _):
            ewv = ew_v[ew_row, pl.ds(g * 16, 16)]
            for t in range(16):
                wgt = ewv[t]
                r = g * 16 + t
                for k in range(D // 16):
                    sl = pl.ds(k * 16, 16)
                    rows_v[r, sl] = rows_v[r, sl] * wgt
            return 0
        lax.fori_loop(0, CHUNK // 16, _grp, 0)

    def _wait_gather(rows_v):
        pltpu.make_async_copy(h_hbm.at[src_v.at[0]], rows_v, gsem).wait()

    def _wait_scatter(rows_v):
        pltpu.make_async_copy(rows_v, acc_sh.at[dst_v.at[0]], ssem).wait()

    nb = lax.select(c == 0, B0, B1)
    base = lax.select(c == 0, s * B0, NS * B0 + s * B1)

    def _block(b, _):
        blk = base + b
        pltpu.sync_copy(src_hbm.at[blk], src_v)
        pltpu.sync_copy(dst_hbm.at[blk], dst_v)
        pltpu.sync_copy(ew_hbm.at[blk], ew_v)

        pltpu.async_copy(h_hbm.at[src_v.at[0]], rows0_v, gsem)

        def _pair(p, _):
            j0 = 2 * p
            j1 = 2 * p + 1
            _wait_gather(rows0_v)

            @pl.when(p > 0)
            def _():
                _wait_scatter(rows1_v)
            pltpu.async_copy(h_hbm.at[src_v.at[j1]], rows1_v, gsem)
            _scale(rows0_v, j0)
            pltpu.async_copy(rows0_v, acc_sh.at[dst_v.at[j0]], ssem,
                             add=True)
            _wait_gather(rows1_v)
            _scale(rows1_v, j1)
            _wait_scatter(rows0_v)

            @pl.when(p < EB // 2 - 1)
            def _():
                pltpu.async_copy(h_hbm.at[src_v.at[j0 + 2]], rows0_v, gsem)
            pltpu.async_copy(rows1_v, acc_sh.at[dst_v.at[j1]], ssem,
                             add=True)
            return 0
        lax.fori_loop(0, EB // 2, _pair, 0)
        _wait_scatter(rows1_v)
        return 0
    lax.fori_loop(0, nb, _block, 0)
    plsc.subcore_barrier()

    pltpu.sync_copy(acc_sh.at[pl.ds(s * RPT, RPT)],
                    out_hbm.at[c, pl.ds(s * RPT, RPT)])


def _prelu(x, a):
    return jnp.where(x >= 0.0, x, a * x)


def _tc1_body(agg_ref, w1_ref, b1_ref, w2_ref, a1_ref, h2_ref):
    agg = agg_ref[0] + agg_ref[1]
    x1 = jnp.dot(agg, w1_ref[...], preferred_element_type=jnp.float32)
    x1 = _prelu(x1 + b1_ref[...], a1_ref[0, 0])
    h2_ref[...] = jnp.dot(x1, w2_ref[...], preferred_element_type=jnp.float32)


def _tc2_body(agg_ref, b2_ref, a2_ref, seq_ref, wfc1_ref, wfc2_ref,
              wfc3_ref, wfc4_ref, a3_ref, x2_ref, feat_ref):
    agg = agg_ref[0] + agg_ref[1]
    x2 = _prelu(agg + b2_ref[...], a2_ref[0, 0])
    x2_ref[...] = x2
    f = jnp.dot(x2, wfc1_ref[...], preferred_element_type=jnp.float32)
    sproj = jnp.dot(seq_ref[...], wfc2_ref[...],
                    preferred_element_type=jnp.float32)
    u = (jnp.dot(sproj, wfc3_ref[0:256, :], preferred_element_type=jnp.float32)
         + jnp.dot(f, wfc3_ref[256:512, :], preferred_element_type=jnp.float32))
    feat_ref[...] = _prelu(
        jnp.dot(u, wfc4_ref[...], preferred_element_type=jnp.float32),
        a3_ref[0, 0])


_R = 400


def _full(shape):
    return pl.BlockSpec(shape, lambda i: (0,) * len(shape))


def _tc1(agg, w1, b1, w2, a1):
    return pl.pallas_call(
        _tc1_body,
        grid=(N // _R,),
        in_specs=[
            pl.BlockSpec((NC, _R, D), lambda i: (0, i, 0)),
            _full((D, 2 * D)), _full((1, 2 * D)), _full((2 * D, D)),
            _full((1, 1)),
        ],
        out_specs=pl.BlockSpec((_R, D), lambda i: (i, 0)),
        out_shape=jax.ShapeDtypeStruct((N, D), jnp.float32),
    )(agg, w1, b1, w2, a1)


def _tc2(agg, b2, a2, seq, wfc1, wfc2, wfc3, wfc4, a3):
    return pl.pallas_call(
        _tc2_body,
        grid=(N // _R,),
        in_specs=[
            pl.BlockSpec((NC, _R, D), lambda i: (0, i, 0)),
            _full((1, D)), _full((1, 1)),
            pl.BlockSpec((_R, D), lambda i: (i, 0)),
            _full((D, 2 * D)), _full((D, 2 * D)), _full((4 * D, D)),
            _full((D, D)), _full((1, 1)),
        ],
        out_specs=[
            pl.BlockSpec((_R, D), lambda i: (i, 0)),
            pl.BlockSpec((_R, D), lambda i: (i, 0)),
        ],
        out_shape=[
            jax.ShapeDtypeStruct((N, D), jnp.float32),
            jax.ShapeDtypeStruct((N, D), jnp.float32),
        ],
    )(agg, b2, a2, seq, wfc1, wfc2, wfc3, wfc4, a3)


def kernel(seq, edge_index, edge_weight, W1, b1, W2, b2, a1, a2, a3,
           Wfc1, Wfc2, Wfc3, Wfc4):
    pad = E_PAD - E
    src = jnp.concatenate(
        [edge_index[0].astype(jnp.int32), jnp.zeros((pad,), jnp.int32)])
    dst = jnp.concatenate(
        [edge_index[1].astype(jnp.int32),
         N + (jnp.arange(pad, dtype=jnp.int32) % (NPAD - N))])
    ew = jnp.concatenate(
        [edge_weight.astype(jnp.float32), jnp.zeros((pad,), jnp.float32)])
    src = src.reshape(TB, EB, CHUNK)
    dst = dst.reshape(TB, EB, CHUNK)
    ew = ew.reshape(TB, EB, CHUNK)

    agg1 = _aggregate(seq, src, dst, ew)
    h2 = _tc1(agg1, W1, b1.reshape(1, -1), W2,
              jnp.asarray(a1, jnp.float32).reshape(1, 1))
    agg2 = _aggregate(h2, src, dst, ew)
    x2, feat1 = _tc2(agg2, b2.reshape(1, -1),
                     jnp.asarray(a2, jnp.float32).reshape(1, 1),
                     seq, Wfc1, Wfc2, Wfc3, Wfc4,
                     jnp.asarray(a3, jnp.float32).reshape(1, 1))
    return (x2, feat1)


# speedup vs baseline: 1.3882x; 1.1044x over previous
import functools
import jax
import jax.numpy as jnp
from jax import lax
from jax.experimental import pallas as pl
from jax.experimental.pallas import tpu as pltpu
from jax.experimental.pallas import tpu_sc as plsc

N = 10000
E = 320000
D = 128

NC, NS = 2, 16
CHUNK = 128
EB = 20
TB = 128
B0, B1 = 6, 2
E_PAD = TB * EB * CHUNK
NPAD = 10240
RPT = NPAD // NS

_mesh = plsc.VectorSubcoreMesh(core_axis_name="c", subcore_axis_name="s")


@functools.partial(
    pl.kernel,
    out_type=jax.ShapeDtypeStruct((NC, NPAD, D), jnp.float32),
    mesh=_mesh,
    scratch_types=[
        pltpu.VMEM((EB, CHUNK), jnp.int32),
        pltpu.VMEM((EB, CHUNK), jnp.int32),
        pltpu.VMEM((EB, CHUNK), jnp.float32),
        pltpu.VMEM((CHUNK, D), jnp.float32),
        pltpu.VMEM((CHUNK, D), jnp.float32),
        pltpu.VMEM_SHARED((NPAD, D), jnp.float32),
        pltpu.SemaphoreType.DMA,
        pltpu.SemaphoreType.DMA,
    ],
)
def _aggregate(h_hbm, src_hbm, dst_hbm, ew_hbm, out_hbm,
               src_v, dst_v, ew_v, rows0_v, rows1_v, acc_sh, gsem, ssem):
    c = lax.axis_index("c")
    s = lax.axis_index("s")

    def _zero_stage(i, _):
        for k in range(D // 16):
            rows0_v[i, pl.ds(k * 16, 16)] = jnp.zeros((16,), jnp.float32)
        return 0
    lax.fori_loop(0, CHUNK, _zero_stage, 0)
    for i in range(RPT // CHUNK):
        pltpu.sync_copy(rows0_v, acc_sh.at[pl.ds(s * RPT + i * CHUNK, CHUNK)])
    plsc.subcore_barrier()

    def _scale(rows_v, ew_row):
        def _grp(g, _):
            ewv = ew_v[ew_row, pl.ds(g * 16, 16)]
            for t in range(16):
                wgt = ewv[t]
                r = g * 16 + t
                for k in range(D // 16):
                    sl = pl.ds(k * 16, 16)
                    rows_v[r, sl] = rows_v[r, sl] * wgt
            return 0
        lax.fori_loop(0, CHUNK // 16, _grp, 0)

    def _wait_gather(rows_v):
        pltpu.make_async_copy(h_hbm.at[src_v.at[0]], rows_v, gsem).wait()

    def _wait_scatter(rows_v):
        pltpu.make_async_copy(rows_v, acc_sh.at[dst_v.at[0]], ssem).wait()

    nb = lax.select(c == 0, B0, B1)
    base = lax.select(c == 0, s * B0, NS * B0 + s * B1)

    def _block(b, _):
        blk = base + b
        pltpu.sync_copy(src_hbm.at[blk], src_v)
        pltpu.sync_copy(dst_hbm.at[blk], dst_v)
        pltpu.sync_copy(ew_hbm.at[blk], ew_v)

        pltpu.async_copy(h_hbm.at[src_v.at[0]], rows0_v, gsem)

        def _pair(p, _):
            j0 = 2 * p
            j1 = 2 * p + 1
            _wait_gather(rows0_v)

            @pl.when(p > 0)
            def _():
                _wait_scatter(rows1_v)
            pltpu.async_copy(h_hbm.at[src_v.at[j1]], rows1_v, gsem)
            _scale(rows0_v, j0)
            pltpu.async_copy(rows0_v, acc_sh.at[dst_v.at[j0]], ssem,
                             add=True)
            _wait_gather(rows1_v)
            _scale(rows1_v, j1)
            _wait_scatter(rows0_v)

            @pl.when(p < EB // 2 - 1)
            def _():
                pltpu.async_copy(h_hbm.at[src_v.at[j0 + 2]], rows0_v, gsem)
            pltpu.async_copy(rows1_v, acc_sh.at[dst_v.at[j1]], ssem,
                             add=True)
            return 0
        lax.fori_loop(0, EB // 2, _pair, 0)
        _wait_scatter(rows1_v)
        return 0
    lax.fori_loop(0, nb, _block, 0)
    plsc.subcore_barrier()

    pltpu.sync_copy(acc_sh.at[pl.ds(s * RPT, RPT)],
                    out_hbm.at[c, pl.ds(s * RPT, RPT)])


def _prelu(x, a):
    return jnp.where(x >= 0.0, x, a * x)


def _tc1_body(agg_ref, w1_ref, b1_ref, w2_ref, a1_ref, h2_ref):
    agg = agg_ref[0] + agg_ref[1]
    x1 = jnp.dot(agg, w1_ref[...], preferred_element_type=jnp.float32)
    x1 = _prelu(x1 + b1_ref[...], a1_ref[0, 0])
    h2_ref[...] = jnp.dot(x1, w2_ref[...], preferred_element_type=jnp.float32)


def _tc2_body(agg_ref, b2_ref, a2_ref, seq_ref, wfc1_ref, wfc2_ref,
              wfc3_ref, wfc4_ref, a3_ref, x2_ref, feat_ref):
    agg = agg_ref[0] + agg_ref[1]
    x2 = _prelu(agg + b2_ref[...], a2_ref[0, 0])
    x2_ref[...] = x2
    f = jnp.dot(x2, wfc1_ref[...], preferred_element_type=jnp.float32)
    sproj = jnp.dot(seq_ref[...], wfc2_ref[...],
                    preferred_element_type=jnp.float32)
    u = (jnp.dot(sproj, wfc3_ref[0:256, :], preferred_element_type=jnp.float32)
         + jnp.dot(f, wfc3_ref[256:512, :], preferred_element_type=jnp.float32))
    feat_ref[...] = _prelu(
        jnp.dot(u, wfc4_ref[...], preferred_element_type=jnp.float32),
        a3_ref[0, 0])


_R = 400


def _full(shape):
    return pl.BlockSpec(shape, lambda i: (0,) * len(shape))


def _tc1(agg, w1, b1, w2, a1):
    return pl.pallas_call(
        _tc1_body,
        grid=(N // _R,),
        in_specs=[
            pl.BlockSpec((NC, _R, D), lambda i: (0, i, 0)),
            _full((D, 2 * D)), _full((1, 2 * D)), _full((2 * D, D)),
            _full((1, 1)),
        ],
        out_specs=pl.BlockSpec((_R, D), lambda i: (i, 0)),
        out_shape=jax.ShapeDtypeStruct((N, D), jnp.float32),
    )(agg, w1, b1, w2, a1)


def _tc2(agg, b2, a2, seq, wfc1, wfc2, wfc3, wfc4, a3):
    return pl.pallas_call(
        _tc2_body,
        grid=(N // _R,),
        in_specs=[
            pl.BlockSpec((NC, _R, D), lambda i: (0, i, 0)),
            _full((1, D)), _full((1, 1)),
            pl.BlockSpec((_R, D), lambda i: (i, 0)),
            _full((D, 2 * D)), _full((D, 2 * D)), _full((4 * D, D)),
            _full((D, D)), _full((1, 1)),
        ],
        out_specs=[
            pl.BlockSpec((_R, D), lambda i: (i, 0)),
            pl.BlockSpec((_R, D), lambda i: (i, 0)),
        ],
        out_shape=[
            jax.ShapeDtypeStruct((N, D), jnp.float32),
            jax.ShapeDtypeStruct((N, D), jnp.float32),
        ],
    )(agg, b2, a2, seq, wfc1, wfc2, wfc3, wfc4, a3)


def kernel(seq, edge_index, edge_weight, W1, b1, W2, b2, a1, a2, a3,
           Wfc1, Wfc2, Wfc3, Wfc4):
    pad = E_PAD - E
    src = jnp.concatenate(
        [edge_index[0].astype(jnp.int32), jnp.zeros((pad,), jnp.int32)])
    dst = jnp.concatenate(
        [edge_index[1].astype(jnp.int32),
         N + (jnp.arange(pad, dtype=jnp.int32) % (NPAD - N))])
    ew = jnp.concatenate(
        [edge_weight.astype(jnp.float32), jnp.zeros((pad,), jnp.float32)])
    src = src.reshape(TB, EB, CHUNK)
    dst = dst.reshape(TB, EB, CHUNK)
    ew = ew.reshape(TB, EB, CHUNK)

    agg1 = _aggregate(seq, src, dst, ew)
    h2 = _tc1(agg1, W1, b1.reshape(1, -1), W2,
              jnp.asarray(a1, jnp.float32).reshape(1, 1))
    agg2 = _aggregate(h2, src, dst, ew)
    x2, feat1 = _tc2(agg2, b2.reshape(1, -1),
                     jnp.asarray(a2, jnp.float32).reshape(1, 1),
                     seq, Wfc1, Wfc2, Wfc3, Wfc4,
                     jnp.asarray(a3, jnp.float32).reshape(1, 1))
    return (x2, feat1)


# speedup vs baseline: 1.5499x; 1.1165x over previous
import functools
import jax
import jax.numpy as jnp
from jax import lax
from jax.experimental import pallas as pl
from jax.experimental.pallas import tpu as pltpu
from jax.experimental.pallas import tpu_sc as plsc

N = 10000
E = 320000
D = 128

NC, NS = 2, 16
CHUNK = 128
EB = 20
TB = 128
B0, B1 = 7, 1
E_PAD = TB * EB * CHUNK
NPAD = 10240
RPT = NPAD // NS

_mesh = plsc.VectorSubcoreMesh(core_axis_name="c", subcore_axis_name="s")


@functools.partial(
    pl.kernel,
    out_type=jax.ShapeDtypeStruct((NC, NPAD, D), jnp.float32),
    mesh=_mesh,
    scratch_types=[
        pltpu.VMEM((EB, CHUNK), jnp.int32),
        pltpu.VMEM((EB, CHUNK), jnp.int32),
        pltpu.VMEM((EB, CHUNK), jnp.float32),
        pltpu.VMEM((CHUNK, D), jnp.float32),
        pltpu.VMEM((CHUNK, D), jnp.float32),
        pltpu.VMEM_SHARED((NPAD, D), jnp.float32),
        pltpu.SemaphoreType.DMA,
        pltpu.SemaphoreType.DMA,
    ],
)
def _aggregate(h_hbm, src_hbm, dst_hbm, ew_hbm, out_hbm,
               src_v, dst_v, ew_v, rows0_v, rows1_v, acc_sh, gsem, ssem):
    c = lax.axis_index("c")
    s = lax.axis_index("s")

    def _zero_stage(i, _):
        for k in range(D // 16):
            rows0_v[i, pl.ds(k * 16, 16)] = jnp.zeros((16,), jnp.float32)
        return 0
    lax.fori_loop(0, CHUNK, _zero_stage, 0)
    for i in range(RPT // CHUNK):
        pltpu.sync_copy(rows0_v, acc_sh.at[pl.ds(s * RPT + i * CHUNK, CHUNK)])
    plsc.subcore_barrier()

    def _scale(rows_v, ew_row):
        def _grp(g, _):
            ewv = ew_v[ew_row, pl.ds(g * 16, 16)]
            for t in range(16):
                wgt = ewv[t]
                r = g * 16 + t
                for k in range(D // 16):
                    sl = pl.ds(k * 16, 16)
                    rows_v[r, sl] = rows_v[r, sl] * wgt
            return 0
        lax.fori_loop(0, CHUNK // 16, _grp, 0)

    def _wait_gather(rows_v):
        pltpu.make_async_copy(h_hbm.at[src_v.at[0]], rows_v, gsem).wait()

    def _wait_scatter(rows_v):
        pltpu.make_async_copy(rows_v, acc_sh.at[dst_v.at[0]], ssem).wait()

    nb = lax.select(c == 0, B0, B1)
    base = lax.select(c == 0, s * B0, NS * B0 + s * B1)

    def _block(b, _):
        blk = base + b
        pltpu.sync_copy(src_hbm.at[blk], src_v)
        pltpu.sync_copy(dst_hbm.at[blk], dst_v)
        pltpu.sync_copy(ew_hbm.at[blk], ew_v)

        pltpu.async_copy(h_hbm.at[src_v.at[0]], rows0_v, gsem)

        def _pair(p, _):
            j0 = 2 * p
            j1 = 2 * p + 1
            _wait_gather(rows0_v)

            @pl.when(p > 0)
            def _():
                _wait_scatter(rows1_v)
            pltpu.async_copy(h_hbm.at[src_v.at[j1]], rows1_v, gsem)
            _scale(rows0_v, j0)
            pltpu.async_copy(rows0_v, acc_sh.at[dst_v.at[j0]], ssem,
                             add=True)
            _wait_gather(rows1_v)
            _scale(rows1_v, j1)
            _wait_scatter(rows0_v)

            @pl.when(p < EB // 2 - 1)
            def _():
                pltpu.async_copy(h_hbm.at[src_v.at[j0 + 2]], rows0_v, gsem)
            pltpu.async_copy(rows1_v, acc_sh.at[dst_v.at[j1]], ssem,
                             add=True)
            return 0
        lax.fori_loop(0, EB // 2, _pair, 0)
        _wait_scatter(rows1_v)
        return 0
    lax.fori_loop(0, nb, _block, 0)
    plsc.subcore_barrier()

    pltpu.sync_copy(acc_sh.at[pl.ds(s * RPT, RPT)],
                    out_hbm.at[c, pl.ds(s * RPT, RPT)])


def _prelu(x, a):
    return jnp.where(x >= 0.0, x, a * x)


def _tc1_body(agg_ref, w1_ref, b1_ref, w2_ref, a1_ref, h2_ref):
    agg = agg_ref[0] + agg_ref[1]
    x1 = jnp.dot(agg, w1_ref[...], preferred_element_type=jnp.float32)
    x1 = _prelu(x1 + b1_ref[...], a1_ref[0, 0])
    h2_ref[...] = jnp.dot(x1, w2_ref[...], preferred_element_type=jnp.float32)


def _tc2_body(agg_ref, b2_ref, a2_ref, seq_ref, wfc1_ref, wfc2_ref,
              wfc3_ref, wfc4_ref, a3_ref, x2_ref, feat_ref):
    agg = agg_ref[0] + agg_ref[1]
    x2 = _prelu(agg + b2_ref[...], a2_ref[0, 0])
    x2_ref[...] = x2
    f = jnp.dot(x2, wfc1_ref[...], preferred_element_type=jnp.float32)
    sproj = jnp.dot(seq_ref[...], wfc2_ref[...],
                    preferred_element_type=jnp.float32)
    u = (jnp.dot(sproj, wfc3_ref[0:256, :], preferred_element_type=jnp.float32)
         + jnp.dot(f, wfc3_ref[256:512, :], preferred_element_type=jnp.float32))
    feat_ref[...] = _prelu(
        jnp.dot(u, wfc4_ref[...], preferred_element_type=jnp.float32),
        a3_ref[0, 0])


_R = 400


def _full(shape):
    return pl.BlockSpec(shape, lambda i: (0,) * len(shape))


def _tc1(agg, w1, b1, w2, a1):
    return pl.pallas_call(
        _tc1_body,
        grid=(N // _R,),
        in_specs=[
            pl.BlockSpec((NC, _R, D), lambda i: (0, i, 0)),
            _full((D, 2 * D)), _full((1, 2 * D)), _full((2 * D, D)),
            _full((1, 1)),
        ],
        out_specs=pl.BlockSpec((_R, D), lambda i: (i, 0)),
        out_shape=jax.ShapeDtypeStruct((N, D), jnp.float32),
    )(agg, w1, b1, w2, a1)


def _tc2(agg, b2, a2, seq, wfc1, wfc2, wfc3, wfc4, a3):
    return pl.pallas_call(
        _tc2_body,
        grid=(N // _R,),
        in_specs=[
            pl.BlockSpec((NC, _R, D), lambda i: (0, i, 0)),
            _full((1, D)), _full((1, 1)),
            pl.BlockSpec((_R, D), lambda i: (i, 0)),
            _full((D, 2 * D)), _full((D, 2 * D)), _full((4 * D, D)),
            _full((D, D)), _full((1, 1)),
        ],
        out_specs=[
            pl.BlockSpec((_R, D), lambda i: (i, 0)),
            pl.BlockSpec((_R, D), lambda i: (i, 0)),
        ],
        out_shape=[
            jax.ShapeDtypeStruct((N, D), jnp.float32),
            jax.ShapeDtypeStruct((N, D), jnp.float32),
        ],
    )(agg, b2, a2, seq, wfc1, wfc2, wfc3, wfc4, a3)


def kernel(seq, edge_index, edge_weight, W1, b1, W2, b2, a1, a2, a3,
           Wfc1, Wfc2, Wfc3, Wfc4):
    pad = E_PAD - E
    src = jnp.concatenate(
        [edge_index[0].astype(jnp.int32), jnp.zeros((pad,), jnp.int32)])
    dst = jnp.concatenate(
        [edge_index[1].astype(jnp.int32),
         N + (jnp.arange(pad, dtype=jnp.int32) % (NPAD - N))])
    ew = jnp.concatenate(
        [edge_weight.astype(jnp.float32), jnp.zeros((pad,), jnp.float32)])
    src = src.reshape(TB, EB, CHUNK)
    dst = dst.reshape(TB, EB, CHUNK)
    ew = ew.reshape(TB, EB, CHUNK)

    agg1 = _aggregate(seq, src, dst, ew)
    h2 = _tc1(agg1, W1, b1.reshape(1, -1), W2,
              jnp.asarray(a1, jnp.float32).reshape(1, 1))
    agg2 = _aggregate(h2, src, dst, ew)
    x2, feat1 = _tc2(agg2, b2.reshape(1, -1),
                     jnp.asarray(a2, jnp.float32).reshape(1, 1),
                     seq, Wfc1, Wfc2, Wfc3, Wfc4,
                     jnp.asarray(a3, jnp.float32).reshape(1, 1))
    return (x2, feat1)
